# Initial kernel scaffold; baseline (speedup 1.0000x reference)
import jax
import jax.numpy as jnp
from jax.experimental import pallas as pl


def kernel(x, edge_index, W1, b1, Wq, bq, Wk, bk, Wr0, br0, Wr1, br1, Wr2, br2, Wr3, br3, wr, alpha, ln_g, ln_b):
    raise NotImplementedError("write your pallas kernel here")


# speedup vs baseline: 2.0065x; 2.0065x over previous
import functools

import jax
import jax.numpy as jnp
from jax import lax
from jax.experimental import pallas as pl
from jax.experimental.pallas import tpu as pltpu

BLK = 512


def _tc_first(x_ref, w1_ref, b1_ref, deg_ref, wr0_ref, br0_ref,
              t0_ref, vs0_ref, head_ref):
    t = jax.nn.relu(jnp.dot(x_ref[...], w1_ref[...],
                            preferred_element_type=jnp.float32) + b1_ref[...])
    t0_ref[...] = t
    deg = deg_ref[...]
    s = jnp.where(deg > 0.0, lax.rsqrt(deg), 0.0)
    vs0_ref[...] = s * t
    head_ref[...] = jnp.dot(t, wr0_ref[...],
                            preferred_element_type=jnp.float32) + br0_ref[...]


def _tc_stats(r_ref, wq_ref, bq_ref, wk_ref, bk_ref, q_ref, kv_ref, misc_ref):
    r = r_ref[...]
    q = jnp.dot(r, wq_ref[...], preferred_element_type=jnp.float32) + bq_ref[...]
    k = jnp.dot(r, wk_ref[...], preferred_element_type=jnp.float32) + bk_ref[...]
    q_ref[...] = q
    kv = lax.dot_general(k, r, (((0,), (0,)), ((), ())),
                         preferred_element_type=jnp.float32)
    ksum = jnp.sum(k, axis=0, keepdims=True)
    sq2 = jnp.sum(q * q)
    sk2 = jnp.sum(k * k)
    misc = jnp.concatenate(
        [ksum, jnp.full((1, 64), sq2, jnp.float32),
         jnp.full((1, 64), sk2, jnp.float32),
         jnp.zeros((5, 64), jnp.float32)], axis=0)

    @pl.when(pl.program_id(0) == 0)
    def _init():
        kv_ref[...] = kv
        misc_ref[...] = misc

    @pl.when(pl.program_id(0) != 0)
    def _acc():
        kv_ref[...] += kv
        misc_ref[...] += misc


def _tc_update(n_real, q_ref, r_ref, acc_ref, deg_ref, kv_ref, misc_ref,
               g_ref, b_ref, alpha_ref, wri_ref, bri_ref, hin_ref,
               rn_ref, vs_ref, hout_ref):
    nf = jnp.float32(n_real)
    misc = misc_ref[...]
    ksum = misc[0:1, :]
    rqn = lax.rsqrt(misc[1, 0])
    rkn = lax.rsqrt(misc[2, 0])
    scale = rqn * rkn
    q = q_ref[...]
    r = r_ref[...]
    attn_num = jnp.dot(q, kv_ref[...] * scale,
                       preferred_element_type=jnp.float32) + nf * r
    normalizer = jnp.sum(q * (ksum * scale), axis=1, keepdims=True)
    attn = attn_num / (normalizer + nf)
    deg = deg_ref[...]
    s = jnp.where(deg > 0.0, lax.rsqrt(deg), 0.0)
    gcn = s * acc_ref[...]
    alpha = alpha_ref[0, 0]
    r1 = alpha * (attn + gcn) + (1.0 - alpha) * r
    mu = jnp.mean(r1, axis=1, keepdims=True)
    var = jnp.mean((r1 - mu) ** 2, axis=1, keepdims=True)
    rn = (r1 - mu) * lax.rsqrt(var + 1e-5) * g_ref[...] + b_ref[...]
    rn_ref[...] = rn
    vs_ref[...] = s * rn
    hout_ref[...] = hin_ref[...] + jnp.dot(
        rn, wri_ref[...], preferred_element_type=jnp.float32) + bri_ref[...]


def _row_spec():
    return pl.BlockSpec((BLK, 64), lambda i: (i, 0))


def _full_spec(shape):
    return pl.BlockSpec(shape, lambda i: tuple(0 for _ in shape))


def kernel(x, edge_index, W1, b1, Wq, bq, Wk, bk, Wr0, br0, Wr1, br1,
           Wr2, br2, Wr3, br3, wr, alpha, ln_g, ln_b):
    n, d_in = x.shape
    npad = ((n + BLK - 1) // BLK) * BLK
    grid = npad // BLK
    f32 = jnp.float32

    xp = jnp.zeros((npad, 64), f32).at[:n, :d_in].set(x)
    w1p = jnp.zeros((64, 64), f32).at[:d_in, :].set(W1)
    row, col = edge_index[0], edge_index[1]

    deg1 = jnp.zeros((npad,), f32).at[col].add(1.0)
    deg = deg1[:, None]
    s_vec = jnp.where(deg1 > 0.0, lax.rsqrt(deg1), 0.0)

    def sc_scatter(vs):
        return jnp.zeros((npad, 64), f32).at[col].add(vs[row])

    b1r = b1[None, :]
    bqr = bq[None, :]
    bkr = bk[None, :]
    gr = ln_g[None, :]
    br = ln_b[None, :]
    alpha_r = jnp.reshape(alpha, (1, 1))
    wrs = [Wr0 * wr[0], Wr1 * wr[1], Wr2 * wr[2], Wr3 * wr[3]]
    brs = [jnp.reshape(br0 * wr[0], (1, 1)), jnp.reshape(br1 * wr[1], (1, 1)),
           jnp.reshape(br2 * wr[2], (1, 1)), jnp.reshape(br3 * wr[3], (1, 1))]

    t0, vs0, head = pl.pallas_call(
        _tc_first,
        grid=(grid,),
        in_specs=[_row_spec(), _full_spec((64, 64)), _full_spec((1, 64)),
                  pl.BlockSpec((BLK, 1), lambda i: (i, 0)),
                  _full_spec((64, 1)), _full_spec((1, 1))],
        out_specs=[_row_spec(), _row_spec(),
                   pl.BlockSpec((BLK, 1), lambda i: (i, 0))],
        out_shape=[jax.ShapeDtypeStruct((npad, 64), f32),
                   jax.ShapeDtypeStruct((npad, 64), f32),
                   jax.ShapeDtypeStruct((npad, 1), f32)],
    )(xp, w1p, b1r, deg, wrs[0], brs[0])

    r_cur = t0
    vs_cur = vs0
    for layer in range(3):
        q, kv, misc = pl.pallas_call(
            _tc_stats,
            grid=(grid,),
            in_specs=[_row_spec(), _full_spec((64, 64)), _full_spec((1, 64)),
                      _full_spec((64, 64)), _full_spec((1, 64))],
            out_specs=[_row_spec(), _full_spec((64, 64)), _full_spec((8, 64))],
            out_shape=[jax.ShapeDtypeStruct((npad, 64), f32),
                       jax.ShapeDtypeStruct((64, 64), f32),
                       jax.ShapeDtypeStruct((8, 64), f32)],
            compiler_params=pltpu.CompilerParams(
                dimension_semantics=("arbitrary",)),
        )(r_cur, Wq, bqr, Wk, bkr)

        acc = sc_scatter(vs_cur)

        r_new, vs_new, head = pl.pallas_call(
            functools.partial(_tc_update, n),
            grid=(grid,),
            in_specs=[_row_spec(), _row_spec(), _row_spec(),
                      pl.BlockSpec((BLK, 1), lambda i: (i, 0)),
                      _full_spec((64, 64)), _full_spec((8, 64)),
                      _full_spec((1, 64)), _full_spec((1, 64)),
                      _full_spec((1, 1)), _full_spec((64, 1)),
                      _full_spec((1, 1)),
                      pl.BlockSpec((BLK, 1), lambda i: (i, 0))],
            out_specs=[_row_spec(), _row_spec(),
                       pl.BlockSpec((BLK, 1), lambda i: (i, 0))],
            out_shape=[jax.ShapeDtypeStruct((npad, 64), f32),
                       jax.ShapeDtypeStruct((npad, 64), f32),
                       jax.ShapeDtypeStruct((npad, 1), f32)],
        )(q, r_cur, acc, deg, kv, misc, gr, br, alpha_r,
          wrs[layer + 1], brs[layer + 1], head)
        r_cur = r_new
        vs_cur = vs_new

    return head[:n]


# speedup vs baseline: 2.0068x; 1.0002x over previous
import functools

import jax
import jax.numpy as jnp
from jax import lax
from jax.experimental import pallas as pl
from jax.experimental.pallas import tpu as pltpu

BLK = 512


def _tc_first(x_ref, w1_ref, b1_ref, deg_ref, wr0_ref, br0_ref,
              t0_ref, vs0_ref, head_ref):
    t = jax.nn.relu(jnp.dot(x_ref[...], w1_ref[...],
                            preferred_element_type=jnp.float32) + b1_ref[...])
    t0_ref[...] = t
    deg = deg_ref[...]
    s = jnp.where(deg > 0.0, lax.rsqrt(deg), 0.0)
    vs0_ref[...] = s * t
    head_ref[...] = jnp.dot(t, wr0_ref[...],
                            preferred_element_type=jnp.float32) + br0_ref[...]


def _tc_stats(r_ref, wq_ref, bq_ref, wk_ref, bk_ref, q_ref, kv_ref, misc_ref):
    r = r_ref[...]
    q = jnp.dot(r, wq_ref[...], preferred_element_type=jnp.float32) + bq_ref[...]
    k = jnp.dot(r, wk_ref[...], preferred_element_type=jnp.float32) + bk_ref[...]
    q_ref[...] = q
    kv = lax.dot_general(k, r, (((0,), (0,)), ((), ())),
                         preferred_element_type=jnp.float32)
    ksum = jnp.sum(k, axis=0, keepdims=True)
    sq2 = jnp.sum(q * q)
    sk2 = jnp.sum(k * k)
    misc = jnp.concatenate(
        [ksum, jnp.full((1, 64), sq2, jnp.float32),
         jnp.full((1, 64), sk2, jnp.float32),
         jnp.zeros((5, 64), jnp.float32)], axis=0)

    @pl.when(pl.program_id(0) == 0)
    def _init():
        kv_ref[...] = kv
        misc_ref[...] = misc

    @pl.when(pl.program_id(0) != 0)
    def _acc():
        kv_ref[...] += kv
        misc_ref[...] += misc


def _tc_update(n_real, q_ref, r_ref, acc_ref, deg_ref, kv_ref, misc_ref,
               g_ref, b_ref, alpha_ref, wri_ref, bri_ref, hin_ref,
               rn_ref, vs_ref, hout_ref):
    nf = jnp.float32(n_real)
    misc = misc_ref[...]
    ksum = misc[0:1, :]
    rqn = lax.rsqrt(misc[1, 0])
    rkn = lax.rsqrt(misc[2, 0])
    scale = rqn * rkn
    q = q_ref[...]
    r = r_ref[...]
    attn_num = jnp.dot(q, kv_ref[...] * scale,
                       preferred_element_type=jnp.float32) + nf * r
    normalizer = jnp.sum(q * (ksum * scale), axis=1, keepdims=True)
    attn = attn_num / (normalizer + nf)
    deg = deg_ref[...]
    s = jnp.where(deg > 0.0, lax.rsqrt(deg), 0.0)
    gcn = s * acc_ref[...]
    alpha = alpha_ref[0, 0]
    r1 = alpha * (attn + gcn) + (1.0 - alpha) * r
    mu = jnp.mean(r1, axis=1, keepdims=True)
    var = jnp.mean((r1 - mu) ** 2, axis=1, keepdims=True)
    rn = (r1 - mu) * lax.rsqrt(var + 1e-5) * g_ref[...] + b_ref[...]
    rn_ref[...] = rn
    vs_ref[...] = s * rn
    hout_ref[...] = hin_ref[...] + jnp.dot(
        rn, wri_ref[...], preferred_element_type=jnp.float32) + bri_ref[...]


def _row_spec():
    return pl.BlockSpec((BLK, 64), lambda i: (i, 0))


def _full_spec(shape):
    return pl.BlockSpec(shape, lambda i: tuple(0 for _ in shape))


def kernel(x, edge_index, W1, b1, Wq, bq, Wk, bk, Wr0, br0, Wr1, br1,
           Wr2, br2, Wr3, br3, wr, alpha, ln_g, ln_b):
    n, d_in = x.shape
    npad = ((n + BLK - 1) // BLK) * BLK
    grid = npad // BLK
    f32 = jnp.float32

    xp = jnp.zeros((npad, 64), f32).at[:n, :d_in].set(x)
    w1p = jnp.zeros((64, 64), f32).at[:d_in, :].set(W1)
    row, col = edge_index[0], edge_index[1]

    deg1 = jnp.zeros((npad,), f32).at[col].add(1.0)
    deg = deg1[:, None]

    def sc_scatter(vs):
        return jnp.zeros((npad, 64), f32).at[col].add(vs[row])

    b1r = b1[None, :]
    bqr = bq[None, :]
    bkr = bk[None, :]
    gr = ln_g[None, :]
    br = ln_b[None, :]
    alpha_r = jnp.reshape(alpha, (1, 1))
    wrs = [Wr0 * wr[0], Wr1 * wr[1], Wr2 * wr[2], Wr3 * wr[3]]
    brs = [jnp.reshape(br0 * wr[0], (1, 1)), jnp.reshape(br1 * wr[1], (1, 1)),
           jnp.reshape(br2 * wr[2], (1, 1)), jnp.reshape(br3 * wr[3], (1, 1))]

    t0, vs0, head = pl.pallas_call(
        _tc_first,
        grid=(grid,),
        in_specs=[_row_spec(), _full_spec((64, 64)), _full_spec((1, 64)),
                  pl.BlockSpec((BLK, 1), lambda i: (i, 0)),
                  _full_spec((64, 1)), _full_spec((1, 1))],
        out_specs=[_row_spec(), _row_spec(),
                   pl.BlockSpec((BLK, 1), lambda i: (i, 0))],
        out_shape=[jax.ShapeDtypeStruct((npad, 64), f32),
                   jax.ShapeDtypeStruct((npad, 64), f32),
                   jax.ShapeDtypeStruct((npad, 1), f32)],
    )(xp, w1p, b1r, deg, wrs[0], brs[0])

    r_cur = t0
    vs_cur = vs0
    for layer in range(3):
        q, kv, misc = pl.pallas_call(
            _tc_stats,
            grid=(grid,),
            in_specs=[_row_spec(), _full_spec((64, 64)), _full_spec((1, 64)),
                      _full_spec((64, 64)), _full_spec((1, 64))],
            out_specs=[_row_spec(), _full_spec((64, 64)), _full_spec((8, 64))],
            out_shape=[jax.ShapeDtypeStruct((npad, 64), f32),
                       jax.ShapeDtypeStruct((64, 64), f32),
                       jax.ShapeDtypeStruct((8, 64), f32)],
            compiler_params=pltpu.CompilerParams(
                dimension_semantics=("arbitrary",)),
        )(r_cur, Wq, bqr, Wk, bkr)

        acc = sc_scatter(vs_cur)

        r_new, vs_new, head = pl.pallas_call(
            functools.partial(_tc_update, n),
            grid=(grid,),
            in_specs=[_row_spec(), _row_spec(), _row_spec(),
                      pl.BlockSpec((BLK, 1), lambda i: (i, 0)),
                      _full_spec((64, 64)), _full_spec((8, 64)),
                      _full_spec((1, 64)), _full_spec((1, 64)),
                      _full_spec((1, 1)), _full_spec((64, 1)),
                      _full_spec((1, 1)),
                      pl.BlockSpec((BLK, 1), lambda i: (i, 0))],
            out_specs=[_row_spec(), _row_spec(),
                       pl.BlockSpec((BLK, 1), lambda i: (i, 0))],
            out_shape=[jax.ShapeDtypeStruct((npad, 64), f32),
                       jax.ShapeDtypeStruct((npad, 64), f32),
                       jax.ShapeDtypeStruct((npad, 1), f32)],
        )(q, r_cur, acc, deg, kv, misc, gr, br, alpha_r,
          wrs[layer + 1], brs[layer + 1], head)
        r_cur = r_new
        vs_cur = vs_new

    return head[:n]
